# baseline (speedup 1.0000x reference)
import functools
import jax
import jax.numpy as jnp
from jax import lax
from jax.experimental import pallas as pl
from jax.experimental.pallas import tpu as pltpu
from jax.experimental.pallas import tpu_sc as plsc

K_SIM = 8
K_CON = 4
KP = 16
NEG_BIG = -1.0e9
POS_BIG = 1.0e9
KILL_NEG = -3.0e9
KILL_POS = 3.0e9
INVALID_THRESH = -0.5e9


def _norm_body(x_ref, xn_ref):
    x = x_ref[...]
    n = jnp.sqrt(jnp.sum(x * x, axis=-1, keepdims=True))
    xn_ref[...] = (x / jnp.maximum(n, 1e-12)).astype(jnp.bfloat16)


def _picks_body(params_ref, xn_ref, idx_ref, w_ref, sim_ref, *, T, Bq,
                k_sim, k_con):
    b = pl.program_id(0)
    qi = pl.program_id(1)
    nkb = T // Bq
    qbase = qi * Bq
    Xnq = xn_ref[0, pl.ds(qbase, Bq), :]
    for kb in range(nkb):
        @pl.when(kb <= qi)
        def _(kb=kb):
            Xk = xn_ref[0, pl.ds(kb * Bq, Bq), :]
            sim_ref[:, kb * Bq:(kb + 1) * Bq] = jax.lax.dot_general(
                Xnq, Xk, (((1,), (1,)), ((), ())),
                preferred_element_type=jnp.float32,
            )
    rows = qbase + jax.lax.broadcasted_iota(jnp.int32, (Bq, 1), 0)
    cols = jax.lax.broadcasted_iota(jnp.int32, (Bq, T), 1)
    valid = cols < rows
    work = jnp.where(valid, sim_ref[...], NEG_BIG)

    alpha = params_ref[1]
    gbase = b * T

    idxs, oks = [], []
    for _ in range(k_sim):
        m = jnp.max(work, axis=1, keepdims=True)
        hit = work == m
        idx = jnp.min(jnp.where(hit, cols, T), axis=1, keepdims=True)
        pick = cols == idx
        oks.append(m > INVALID_THRESH)
        idxs.append(idx)
        work = jnp.where(pick, KILL_NEG, work)
    deg_sim = functools.reduce(
        lambda a, c: a + c.astype(jnp.float32), oks, jnp.zeros((Bq, 1)))

    simc = jnp.where(work > INVALID_THRESH, work, POS_BIG)
    mcon = jnp.maximum(0, k_con - (T - rows))
    oks_c, idxs_c = [], []
    for j in range(k_con):
        mn = jnp.min(simc, axis=1, keepdims=True)
        hit = simc == mn
        idx = jnp.min(jnp.where(hit, cols, T), axis=1, keepdims=True)
        pick = cols == idx
        oks_c.append((mn < -INVALID_THRESH) & (j < mcon))
        idxs_c.append(idx)
        simc = jnp.where(pick, KILL_POS, simc)
    deg_con = functools.reduce(
        lambda a, c: a + c.astype(jnp.float32), oks_c, jnp.zeros((Bq, 1)))

    w_sim = alpha / jnp.maximum(deg_sim, 1.0)
    w_con = (1.0 - alpha) / jnp.maximum(deg_con, 1.0)
    gidxs = [jnp.where(ok, idx + gbase, 0) for ok, idx in zip(oks, idxs)]
    gidxs += [jnp.where(ok, idx + gbase, 0) for ok, idx in zip(oks_c, idxs_c)]
    ws = [jnp.where(ok, w_sim, 0.0) for ok in oks]
    ws += [jnp.where(ok, w_con, 0.0) for ok in oks_c]
    npad = KP - len(gidxs)
    gidxs += [jnp.zeros((Bq, 1), jnp.int32)] * npad
    ws += [jnp.zeros((Bq, 1), jnp.float32)] * npad
    idx_ref[0] = jnp.concatenate(gidxs, axis=1)
    w_ref[0] = jnp.concatenate(ws, axis=1)


def _make_sc_agg(BT, D, NC, NW, L, C):
    Q = BT // NW

    def _sc_agg(x_hbm, gidx_hbm, wexp_hbm, out_hbm, idx_v, rows_v, wv_v,
                out_v, sem):
        wid = lax.axis_index("s") * NC + lax.axis_index("c")
        base = wid * Q

        def chunk(ci, carry):
            qb = base + ci * C
            pltpu.sync_copy(gidx_hbm.at[pl.ds(qb * KP, C * KP)], idx_v)
            pltpu.sync_copy(wexp_hbm.at[pl.ds(qb * KP, C * KP)], wv_v)
            pltpu.async_copy(x_hbm.at[idx_v], rows_v, sem).wait()
            for qq in range(C):
                wvecs = [wv_v[qq * KP + k] for k in range(KP)]

                def gbody(g, carry2, qq=qq, wvecs=wvecs):
                    s = pl.ds(g * L, L)
                    acc = rows_v[qq * KP, s] * wvecs[0]
                    for k in range(1, KP):
                        acc = acc + rows_v[qq * KP + k, s] * wvecs[k]
                    out_v[qq, s] = acc
                    return carry2

                lax.fori_loop(0, D // L, gbody, 0)
            pltpu.sync_copy(out_v, out_hbm.at[pl.ds(qb, C)])
            return carry

        lax.fori_loop(0, Q // C, chunk, 0)

    return _sc_agg


def _epi_body(params_ref, gain_ref, bias_ref, x_ref, ctx_ref, out_ref):
    mix = params_ref[0]
    scale = params_ref[2]
    blended = mix * x_ref[...] + (1.0 - mix) * ctx_ref[...]
    t = blended * gain_ref[...] + bias_ref[...]
    g = 0.5 * t * (1.0 + jax.lax.erf(t * 0.7071067811865476))
    out_ref[...] = g * scale


def kernel(x, gain, bias, log_mix, log_alpha, log_scale):
    B, T, D = x.shape
    BT = B * T
    Bq = 256
    k_sim = min(K_SIM, T - 1)
    k_con = min(K_CON, max(0, T - 1 - k_sim))

    mix = jax.nn.sigmoid(log_mix)
    alpha = jax.nn.sigmoid(log_alpha)
    scale = jax.nn.softplus(log_scale) + 0.01
    params = jnp.stack([mix, alpha, scale]).astype(jnp.float32)

    xn = pl.pallas_call(
        _norm_body,
        grid=(BT // Bq,),
        in_specs=[pl.BlockSpec((Bq, D), lambda i: (i, 0))],
        out_specs=pl.BlockSpec((Bq, D), lambda i: (i, 0)),
        out_shape=jax.ShapeDtypeStruct((BT, D), jnp.bfloat16),
    )(x.reshape(BT, D)).reshape(B, T, D)

    gidx, w = pl.pallas_call(
        functools.partial(_picks_body, T=T, Bq=Bq, k_sim=k_sim, k_con=k_con),
        grid=(B, T // Bq),
        in_specs=[
            pl.BlockSpec(memory_space=pltpu.SMEM),
            pl.BlockSpec((1, T, D), lambda b, q: (b, 0, 0)),
        ],
        out_specs=[
            pl.BlockSpec((1, Bq, KP), lambda b, q: (b, q, 0)),
            pl.BlockSpec((1, Bq, KP), lambda b, q: (b, q, 0)),
        ],
        out_shape=[
            jax.ShapeDtypeStruct((B, T, KP), jnp.int32),
            jax.ShapeDtypeStruct((B, T, KP), jnp.float32),
        ],
        scratch_shapes=[pltpu.VMEM((Bq, T), jnp.float32)],
    )(params, xn)

    info = plsc.get_sparse_core_info()
    NC, NS, L = info.num_cores, info.num_subcores, info.num_lanes
    NW = NC * NS
    C = 2
    gidx1 = gidx.reshape(BT * KP)
    wexp = jnp.broadcast_to(w.reshape(BT * KP, 1), (BT * KP, L))

    mesh = plsc.VectorSubcoreMesh(core_axis_name="c", subcore_axis_name="s")
    ctx = pl.kernel(
        _make_sc_agg(BT, D, NC, NW, L, C),
        mesh=mesh,
        out_type=jax.ShapeDtypeStruct((BT, D), jnp.float32),
        scratch_types=[
            pltpu.VMEM((C * KP,), jnp.int32),
            pltpu.VMEM((C * KP, D), jnp.float32),
            pltpu.VMEM((C * KP, L), jnp.float32),
            pltpu.VMEM((C, D), jnp.float32),
            pltpu.SemaphoreType.DMA,
        ],
    )(x.reshape(BT, D), gidx1, wexp)

    delta = pl.pallas_call(
        _epi_body,
        grid=(BT // Bq,),
        in_specs=[
            pl.BlockSpec(memory_space=pltpu.SMEM),
            pl.BlockSpec((1, D), lambda i: (0, 0)),
            pl.BlockSpec((1, D), lambda i: (0, 0)),
            pl.BlockSpec((Bq, D), lambda i: (i, 0)),
            pl.BlockSpec((Bq, D), lambda i: (i, 0)),
        ],
        out_specs=pl.BlockSpec((Bq, D), lambda i: (i, 0)),
        out_shape=jax.ShapeDtypeStruct((BT, D), jnp.float32),
    )(params, gain.reshape(1, D), bias.reshape(1, D), x.reshape(BT, D), ctx)

    return delta.reshape(B, T, D)


# speedup vs baseline: 9.2800x; 9.2800x over previous
import functools
import jax
import jax.numpy as jnp
from jax.experimental import pallas as pl
from jax.experimental.pallas import tpu as pltpu

K_SIM = 8
K_CON = 4
NEG_BIG = -1.0e9
POS_BIG = 1.0e9
KILL_NEG = -3.0e9
KILL_POS = 3.0e9
INVALID_THRESH = -0.5e9


def _norm_body(x_ref, xn_ref, xb_ref):
    x = x_ref[...]
    n = jnp.sqrt(jnp.sum(x * x, axis=-1, keepdims=True))
    xn_ref[...] = (x / jnp.maximum(n, 1e-12)).astype(jnp.bfloat16)
    xb_ref[...] = x.astype(jnp.bfloat16)


def _main_body(params_ref, gain_ref, bias_ref, xn_ref, xb_ref, xq_ref,
               out_ref, sim_ref, wb_ref, acc_ref, *, T, Tp, qoff, Bq,
               k_sim, k_con):
    qi = pl.program_id(1)
    nkb = Tp // Bq
    qbase = (qi + qoff) * Bq
    D = xb_ref.shape[2]

    Xnq = xn_ref[0, pl.ds(qbase, Bq), :]
    for kb in range(nkb):
        @pl.when(kb <= qi + qoff)
        def _(kb=kb):
            Xk = xn_ref[0, pl.ds(kb * Bq, Bq), :]
            sim_ref[:, kb * Bq:(kb + 1) * Bq] = jax.lax.dot_general(
                Xnq, Xk, (((1,), (1,)), ((), ())),
                preferred_element_type=jnp.float32,
            )
    rows = qbase + jax.lax.broadcasted_iota(jnp.int32, (Bq, 1), 0)
    cols = jax.lax.broadcasted_iota(jnp.int32, (Bq, Tp), 1)
    valid = cols < rows
    work = jnp.where(valid, sim_ref[...], NEG_BIG)

    alpha = params_ref[1]

    deg_sim = jnp.zeros((Bq, 1), jnp.float32)
    for _ in range(k_sim):
        m = jnp.max(work, axis=1, keepdims=True)
        deg_sim += (m > INVALID_THRESH).astype(jnp.float32)
        work = jnp.where(work == m, KILL_NEG, work)
    m_sim = (work == KILL_NEG) & valid
    w_sim = alpha / jnp.maximum(deg_sim, 1.0)
    wb_ref[...] = jnp.where(m_sim, w_sim, 0.0).astype(jnp.bfloat16)

    if k_con > 0 and nkb == T // Bq:
        @pl.when(qi + qoff == nkb - 1)
        def _():
            simc = jnp.where(work > INVALID_THRESH, work, POS_BIG)
            mcon = jnp.maximum(0, k_con - (T - rows))
            m_con = jnp.zeros((Bq, Tp), jnp.bool_)
            deg_con = jnp.zeros((Bq, 1), jnp.float32)
            sc = simc
            for j in range(k_con):
                mn = jnp.min(sc, axis=1, keepdims=True)
                ok = (mn < -INVALID_THRESH) & (j < mcon)
                hit = sc == mn
                m_con = m_con | (hit & ok)
                deg_con += ok.astype(jnp.float32)
                sc = jnp.where(hit, KILL_POS, sc)
            w_con = (1.0 - alpha) / jnp.maximum(deg_con, 1.0)
            wb_ref[...] += jnp.where(m_con, w_con, 0.0).astype(jnp.bfloat16)

    acc_ref[...] = jnp.zeros((Bq, D), jnp.float32)
    for kb in range(nkb):
        @pl.when(kb <= qi + qoff)
        def _(kb=kb):
            A = wb_ref[:, kb * Bq:(kb + 1) * Bq]
            Xk = xb_ref[0, pl.ds(kb * Bq, Bq), :]
            acc_ref[...] += jax.lax.dot_general(
                A, Xk, (((1,), (0,)), ((), ())),
                preferred_element_type=jnp.float32,
            )

    mix = params_ref[0]
    scale = params_ref[2]
    blended = mix * xq_ref[0] + (1.0 - mix) * acc_ref[...]
    t = blended * gain_ref[...] + bias_ref[...]
    g = 0.5 * t * (1.0 + jax.lax.erf(t * 0.7071067811865476))
    out_ref[0] = g * scale


def kernel(x, gain, bias, log_mix, log_alpha, log_scale):
    B, T, D = x.shape
    Bq = 256
    k_sim = min(K_SIM, T - 1)
    k_con = min(K_CON, max(0, T - 1 - k_sim))

    mix = jax.nn.sigmoid(log_mix)
    alpha = jax.nn.sigmoid(log_alpha)
    scale = jax.nn.softplus(log_scale) + 0.01
    params = jnp.stack([mix, alpha, scale]).astype(jnp.float32)

    xn, xb = pl.pallas_call(
        _norm_body,
        grid=(B * T // Bq,),
        in_specs=[pl.BlockSpec((Bq, D), lambda i: (i, 0))],
        out_specs=[
            pl.BlockSpec((Bq, D), lambda i: (i, 0)),
            pl.BlockSpec((Bq, D), lambda i: (i, 0)),
        ],
        out_shape=[
            jax.ShapeDtypeStruct((B * T, D), jnp.bfloat16),
            jax.ShapeDtypeStruct((B * T, D), jnp.bfloat16),
        ],
    )(x.reshape(B * T, D))
    xn = xn.reshape(B, T, D)
    xb = xb.reshape(B, T, D)

    nqb = T // Bq
    halves = []
    for qoff, nq in ((0, nqb // 2), (nqb // 2, nqb - nqb // 2)):
        Tp = (qoff + nq) * Bq
        half = pl.pallas_call(
            functools.partial(_main_body, T=T, Tp=Tp, qoff=qoff, Bq=Bq,
                              k_sim=k_sim, k_con=k_con),
            grid=(B, nq),
            in_specs=[
                pl.BlockSpec(memory_space=pltpu.SMEM),
                pl.BlockSpec((1, D), lambda b, q: (0, 0)),
                pl.BlockSpec((1, D), lambda b, q: (0, 0)),
                pl.BlockSpec((1, Tp, D), lambda b, q: (b, 0, 0)),
                pl.BlockSpec((1, Tp, D), lambda b, q: (b, 0, 0)),
                pl.BlockSpec((1, Bq, D),
                             lambda b, q, qoff=qoff: (b, q + qoff, 0)),
            ],
            out_specs=pl.BlockSpec((1, Bq, D), lambda b, q: (b, q, 0)),
            out_shape=jax.ShapeDtypeStruct((B, nq * Bq, D), jnp.float32),
            scratch_shapes=[
                pltpu.VMEM((Bq, Tp), jnp.float32),
                pltpu.VMEM((Bq, Tp), jnp.bfloat16),
                pltpu.VMEM((Bq, D), jnp.float32),
            ],
        )(params, gain.reshape(1, D), bias.reshape(1, D), xn, xb, x)
        halves.append(half)

    return jnp.concatenate(halves, axis=1)


# speedup vs baseline: 10.2953x; 1.1094x over previous
import functools
import jax
import jax.numpy as jnp
from jax.experimental import pallas as pl
from jax.experimental.pallas import tpu as pltpu

K_SIM = 8
K_CON = 4
NEG_BIG = -1.0e9
POS_BIG = 1.0e9
KILL_NEG = -3.0e9
KILL_POS = 3.0e9
INVALID_THRESH = -0.5e9


def _norm_body(x_ref, xn_ref, xb_ref):
    x = x_ref[...]
    n = jnp.sqrt(jnp.sum(x * x, axis=-1, keepdims=True))
    xn_ref[...] = (x / jnp.maximum(n, 1e-12)).astype(jnp.bfloat16)
    xb_ref[...] = x.astype(jnp.bfloat16)


def _main_body(params_ref, gain_ref, bias_ref, xn_ref, xb_ref, xq_ref,
               out_ref, sim_ref, wb_ref, acc_ref, *, T, Bq, k_sim, k_con):
    qi = pl.program_id(1)
    nkb = T // Bq
    qbase = qi * Bq
    D = xb_ref.shape[2]

    Xnq = xn_ref[0, pl.ds(qbase, Bq), :]
    for kb in range(nkb):
        @pl.when(kb <= qi)
        def _(kb=kb):
            Xk = xn_ref[0, pl.ds(kb * Bq, Bq), :]
            sim_ref[:, kb * Bq:(kb + 1) * Bq] = jax.lax.dot_general(
                Xnq, Xk, (((1,), (1,)), ((), ())),
                preferred_element_type=jnp.float32,
            )
    rows = qbase + jax.lax.broadcasted_iota(jnp.int32, (Bq, 1), 0)
    cols = jax.lax.broadcasted_iota(jnp.int32, (Bq, T), 1)
    valid = cols < rows
    work = jnp.where(valid, sim_ref[...], NEG_BIG)

    alpha = params_ref[1]

    deg_sim = jnp.zeros((Bq, 1), jnp.float32)
    for _ in range(k_sim):
        m = jnp.max(work, axis=1, keepdims=True)
        deg_sim += (m > INVALID_THRESH).astype(jnp.float32)
        work = jnp.where(work == m, KILL_NEG, work)
    m_sim = (work == KILL_NEG) & valid
    w_sim = alpha / jnp.maximum(deg_sim, 1.0)
    wb_ref[...] = jnp.where(m_sim, w_sim, 0.0).astype(jnp.bfloat16)

    if k_con > 0:
        @pl.when(qi == nkb - 1)
        def _():
            simc = jnp.where(work > INVALID_THRESH, work, POS_BIG)
            mcon = jnp.maximum(0, k_con - (T - rows))
            m_con = jnp.zeros((Bq, T), jnp.bool_)
            deg_con = jnp.zeros((Bq, 1), jnp.float32)
            sc = simc
            for j in range(k_con):
                mn = jnp.min(sc, axis=1, keepdims=True)
                ok = (mn < -INVALID_THRESH) & (j < mcon)
                hit = sc == mn
                m_con = m_con | (hit & ok)
                deg_con += ok.astype(jnp.float32)
                sc = jnp.where(hit, KILL_POS, sc)
            w_con = (1.0 - alpha) / jnp.maximum(deg_con, 1.0)
            wb_ref[...] += jnp.where(m_con, w_con, 0.0).astype(jnp.bfloat16)

    acc_ref[...] = jnp.zeros((Bq, D), jnp.float32)
    for kb in range(nkb):
        @pl.when(kb <= qi)
        def _(kb=kb):
            A = wb_ref[:, kb * Bq:(kb + 1) * Bq]
            Xk = xb_ref[0, pl.ds(kb * Bq, Bq), :]
            acc_ref[...] += jax.lax.dot_general(
                A, Xk, (((1,), (0,)), ((), ())),
                preferred_element_type=jnp.float32,
            )

    mix = params_ref[0]
    scale = params_ref[2]
    blended = mix * xq_ref[0] + (1.0 - mix) * acc_ref[...]
    t = blended * gain_ref[...] + bias_ref[...]
    g = 0.5 * t * (1.0 + jax.lax.erf(t * 0.7071067811865476))
    out_ref[0] = g * scale


def kernel(x, gain, bias, log_mix, log_alpha, log_scale):
    B, T, D = x.shape
    Bq = 256
    k_sim = min(K_SIM, T - 1)
    k_con = min(K_CON, max(0, T - 1 - k_sim))

    mix = jax.nn.sigmoid(log_mix)
    alpha = jax.nn.sigmoid(log_alpha)
    scale = jax.nn.softplus(log_scale) + 0.01
    params = jnp.stack([mix, alpha, scale]).astype(jnp.float32)

    xn, xb = pl.pallas_call(
        _norm_body,
        grid=(B * T // Bq,),
        in_specs=[pl.BlockSpec((Bq, D), lambda i: (i, 0))],
        out_specs=[
            pl.BlockSpec((Bq, D), lambda i: (i, 0)),
            pl.BlockSpec((Bq, D), lambda i: (i, 0)),
        ],
        out_shape=[
            jax.ShapeDtypeStruct((B * T, D), jnp.bfloat16),
            jax.ShapeDtypeStruct((B * T, D), jnp.bfloat16),
        ],
        compiler_params=pltpu.CompilerParams(
            dimension_semantics=("parallel",)),
    )(x.reshape(B * T, D))
    xn = xn.reshape(B, T, D)
    xb = xb.reshape(B, T, D)

    delta = pl.pallas_call(
        functools.partial(_main_body, T=T, Bq=Bq, k_sim=k_sim, k_con=k_con),
        grid=(B, T // Bq),
        in_specs=[
            pl.BlockSpec(memory_space=pltpu.SMEM),
            pl.BlockSpec((1, D), lambda b, q: (0, 0)),
            pl.BlockSpec((1, D), lambda b, q: (0, 0)),
            pl.BlockSpec((1, T, D), lambda b, q: (b, 0, 0)),
            pl.BlockSpec((1, T, D), lambda b, q: (b, 0, 0)),
            pl.BlockSpec((1, Bq, D), lambda b, q: (b, q, 0)),
        ],
        out_specs=pl.BlockSpec((1, Bq, D), lambda b, q: (b, q, 0)),
        out_shape=jax.ShapeDtypeStruct((B, T, D), jnp.float32),
        scratch_shapes=[
            pltpu.VMEM((Bq, T), jnp.float32),
            pltpu.VMEM((Bq, T), jnp.bfloat16),
            pltpu.VMEM((Bq, D), jnp.float32),
        ],
        compiler_params=pltpu.CompilerParams(
            dimension_semantics=("parallel", "arbitrary")),
    )(params, gain.reshape(1, D), bias.reshape(1, D), xn, xb, x)

    return delta


# speedup vs baseline: 10.3069x; 1.0011x over previous
import functools
import jax
import jax.numpy as jnp
from jax.experimental import pallas as pl
from jax.experimental.pallas import tpu as pltpu

K_SIM = 8
K_CON = 4
NEG_BIG = -1.0e9
POS_BIG = 1.0e9
KILL_NEG = -3.0e9
KILL_POS = 3.0e9
INVALID_THRESH = -0.5e9


def _norm_body(x_ref, xn_ref, xb_ref):
    x = x_ref[...]
    n = jnp.sqrt(jnp.sum(x * x, axis=-1, keepdims=True))
    xn_ref[...] = (x / jnp.maximum(n, 1e-12)).astype(jnp.bfloat16)
    xb_ref[...] = x.astype(jnp.bfloat16)


def _main_body(params_ref, gain_ref, bias_ref, xn_ref, xb_ref, xq_ref,
               out_ref, sim_ref, wb_ref, acc_ref, *, T, Bq, k_sim, k_con):
    qi = pl.program_id(1)
    nkb = T // Bq
    qbase = qi * Bq
    D = xb_ref.shape[2]

    Xnq = xn_ref[0, pl.ds(qbase, Bq), :]
    for kb in range(nkb):
        @pl.when(kb <= qi)
        def _(kb=kb):
            Xk = xn_ref[0, pl.ds(kb * Bq, Bq), :]
            sim_ref[:, kb * Bq:(kb + 1) * Bq] = jax.lax.dot_general(
                Xnq, Xk, (((1,), (1,)), ((), ())),
                preferred_element_type=jnp.float32,
            )
    rows = qbase + jax.lax.broadcasted_iota(jnp.int32, (Bq, 1), 0)
    cols = jax.lax.broadcasted_iota(jnp.int32, (Bq, T), 1)
    valid = cols < rows
    work = jnp.where(valid, sim_ref[...], NEG_BIG)

    alpha = params_ref[1]

    deg_sim = jnp.zeros((Bq, 1), jnp.float32)
    for _ in range(k_sim):
        m = jnp.max(work, axis=1, keepdims=True)
        deg_sim += (m > INVALID_THRESH).astype(jnp.float32)
        work = jnp.where(work == m, KILL_NEG, work)
    m_sim = (work == KILL_NEG) & valid
    w_sim = alpha / jnp.maximum(deg_sim, 1.0)
    wb_ref[...] = jnp.where(m_sim, w_sim, 0.0).astype(jnp.bfloat16)

    if k_con > 0:
        @pl.when(qi == nkb - 1)
        def _():
            simc = jnp.where(work > INVALID_THRESH, work, POS_BIG)
            mcon = jnp.maximum(0, k_con - (T - rows))
            m_con = jnp.zeros((Bq, T), jnp.bool_)
            deg_con = jnp.zeros((Bq, 1), jnp.float32)
            sc = simc
            for j in range(k_con):
                mn = jnp.min(sc, axis=1, keepdims=True)
                ok = (mn < -INVALID_THRESH) & (j < mcon)
                hit = sc == mn
                m_con = m_con | (hit & ok)
                deg_con += ok.astype(jnp.float32)
                sc = jnp.where(hit, KILL_POS, sc)
            w_con = (1.0 - alpha) / jnp.maximum(deg_con, 1.0)
            wb_ref[...] += jnp.where(m_con, w_con, 0.0).astype(jnp.bfloat16)

    acc_ref[...] = jnp.zeros((Bq, D), jnp.float32)
    for kb in range(nkb):
        @pl.when(kb <= qi)
        def _(kb=kb):
            A = wb_ref[:, kb * Bq:(kb + 1) * Bq]
            Xk = xb_ref[0, pl.ds(kb * Bq, Bq), :]
            acc_ref[...] += jax.lax.dot_general(
                A, Xk, (((1,), (0,)), ((), ())),
                preferred_element_type=jnp.float32,
            )

    mix = params_ref[0]
    scale = params_ref[2]
    blended = mix * xq_ref[0] + (1.0 - mix) * acc_ref[...]
    t = blended * gain_ref[...] + bias_ref[...]
    g = 0.5 * t * (1.0 + jax.lax.erf(t * 0.7071067811865476))
    out_ref[0] = g * scale


def kernel(x, gain, bias, log_mix, log_alpha, log_scale):
    B, T, D = x.shape
    Bq = 256
    k_sim = min(K_SIM, T - 1)
    k_con = min(K_CON, max(0, T - 1 - k_sim))

    mix = jax.nn.sigmoid(log_mix)
    alpha = jax.nn.sigmoid(log_alpha)
    scale = jax.nn.softplus(log_scale) + 0.01
    params = jnp.stack([mix, alpha, scale]).astype(jnp.float32)

    xn, xb = pl.pallas_call(
        _norm_body,
        grid=(B * T // Bq,),
        in_specs=[pl.BlockSpec((Bq, D), lambda i: (i, 0))],
        out_specs=[
            pl.BlockSpec((Bq, D), lambda i: (i, 0)),
            pl.BlockSpec((Bq, D), lambda i: (i, 0)),
        ],
        out_shape=[
            jax.ShapeDtypeStruct((B * T, D), jnp.bfloat16),
            jax.ShapeDtypeStruct((B * T, D), jnp.bfloat16),
        ],
    )(x.reshape(B * T, D))
    xn = xn.reshape(B, T, D)
    xb = xb.reshape(B, T, D)

    delta = pl.pallas_call(
        functools.partial(_main_body, T=T, Bq=Bq, k_sim=k_sim, k_con=k_con),
        grid=(B, T // Bq),
        in_specs=[
            pl.BlockSpec(memory_space=pltpu.SMEM),
            pl.BlockSpec((1, D), lambda b, q: (0, 0)),
            pl.BlockSpec((1, D), lambda b, q: (0, 0)),
            pl.BlockSpec((1, T, D), lambda b, q: (b, 0, 0)),
            pl.BlockSpec((1, T, D), lambda b, q: (b, 0, 0)),
            pl.BlockSpec((1, Bq, D), lambda b, q: (b, q, 0)),
        ],
        out_specs=pl.BlockSpec((1, Bq, D), lambda b, q: (b, q, 0)),
        out_shape=jax.ShapeDtypeStruct((B, T, D), jnp.float32),
        scratch_shapes=[
            pltpu.VMEM((Bq, T), jnp.float32),
            pltpu.VMEM((Bq, T), jnp.bfloat16),
            pltpu.VMEM((Bq, D), jnp.float32),
        ],
    )(params, gain.reshape(1, D), bias.reshape(1, D), xn, xb, x)

    return delta
